# baseline (speedup 1.0000x reference)
import jax
import jax.numpy as jnp
from jax.experimental import pallas as pl
from jax.experimental.pallas import tpu as pltpu

_BLK = 1024
_K = 9


def _body(xa_ref, xb_ref, w0_ref, w1_ref, b_ref, out_ref, g_scr):
    s = pl.program_id(0)

    XA = xa_ref[...]
    g_scr[s % 2] = jax.lax.dot_general(
        XA, XA, (((1,), (1,)), ((), ())), preferred_element_type=jnp.float32
    )

    X = xb_ref[...]
    G = g_scr[(s + 1) % 2]
    sq = jnp.sum(X * X, axis=1)
    d2 = sq[:, None] + sq[None, :] - 2.0 * G

    ii = jax.lax.broadcasted_iota(jnp.int32, (_BLK, _BLK), 0)
    jj = jax.lax.broadcasted_iota(jnp.int32, (_BLK, _BLK), 1)

    is_last = s == 4
    cross = jnp.logical_and(is_last, jnp.logical_xor(ii == _BLK - 1, jj == _BLK - 1))
    d2 = jnp.where(cross, jnp.inf, d2)

    m = jnp.full((_BLK, 1), -jnp.inf, dtype=jnp.float32)
    for _ in range(_K):
        m = jnp.min(jnp.where(d2 > m, d2, jnp.inf), axis=1, keepdims=True)

    dead = jnp.logical_or(cross, ii == jj)
    Af = jnp.where(
        jnp.logical_and(d2 <= m, jnp.logical_not(dead)), 1.0, 0.0
    )

    deg = jnp.sum(Af, axis=0)
    dinv = jnp.where(deg > 0, jax.lax.rsqrt(jnp.maximum(deg, 1e-12)), 0.0)
    An = (-dinv[:, None] * Af) * dinv[None, :]

    Tx1 = jax.lax.dot_general(
        An, X, (((1,), (0,)), ((), ())), preferred_element_type=jnp.float32
    )
    out = (
        jax.lax.dot_general(
            X, w0_ref[...], (((1,), (1,)), ((), ())),
            preferred_element_type=jnp.float32,
        )
        + jax.lax.dot_general(
            Tx1, w1_ref[...], (((1,), (1,)), ((), ())),
            preferred_element_type=jnp.float32,
        )
        + b_ref[...]
    )
    out_ref[...] = out


def kernel(x, W0, W1, b):
    Bn, Cn, Hn, Wn = x.shape
    n = Bn * Hn * Wn
    x_f = jnp.transpose(x, (0, 2, 3, 1)).reshape(n, Cn)
    out = pl.pallas_call(
        _body,
        grid=(Bn + 1,),
        in_specs=[
            pl.BlockSpec((_BLK, Cn), lambda s: (jnp.minimum(s, 3), 0)),
            pl.BlockSpec((_BLK, Cn), lambda s: (jnp.maximum(s - 1, 0), 0)),
            pl.BlockSpec((Cn, Cn), lambda s: (0, 0)),
            pl.BlockSpec((Cn, Cn), lambda s: (0, 0)),
            pl.BlockSpec((1, Cn), lambda s: (0, 0)),
        ],
        out_specs=pl.BlockSpec((_BLK, Cn), lambda s: (jnp.maximum(s - 1, 0), 0)),
        out_shape=jax.ShapeDtypeStruct((n, Cn), jnp.float32),
        scratch_shapes=[pltpu.VMEM((2, _BLK, _BLK), jnp.float32)],
    )(x_f, x_f, W0, W1, b.reshape(1, Cn))
    return out


# speedup vs baseline: 1.3707x; 1.3707x over previous
import jax
import jax.numpy as jnp
from jax.experimental import pallas as pl

_BLK = 1024
_K = 9


def _body(x_ref, w0_ref, w1_ref, b_ref, out_ref):
    pid = pl.program_id(0)
    X = x_ref[...]
    sq = jnp.sum(X * X, axis=1)
    G = jax.lax.dot_general(
        X, X, (((1,), (1,)), ((), ())), preferred_element_type=jnp.float32
    )
    d2 = sq[:, None] + sq[None, :] - 2.0 * G

    ii = jax.lax.broadcasted_iota(jnp.int32, (_BLK, _BLK), 0)
    jj = jax.lax.broadcasted_iota(jnp.int32, (_BLK, _BLK), 1)

    is_last = pid == 3
    cross = jnp.logical_and(is_last, jnp.logical_xor(ii == _BLK - 1, jj == _BLK - 1))
    d2 = jnp.where(cross, jnp.inf, d2)

    m = jnp.full((_BLK, 1), -jnp.inf, dtype=jnp.float32)
    for _ in range(_K):
        m = jnp.min(jnp.where(d2 > m, d2, jnp.inf), axis=1, keepdims=True)

    dead = jnp.logical_or(cross, ii == jj)
    Af = jnp.where(
        jnp.logical_and(d2 <= m, jnp.logical_not(dead)), 1.0, 0.0
    )

    deg = jnp.sum(Af, axis=0)
    dinv = jnp.where(deg > 0, jax.lax.rsqrt(jnp.maximum(deg, 1e-12)), 0.0)
    An = (-dinv[:, None] * Af) * dinv[None, :]

    Tx1 = jax.lax.dot_general(
        An, X, (((1,), (0,)), ((), ())), preferred_element_type=jnp.float32
    )
    out = (
        jax.lax.dot_general(
            X, w0_ref[...], (((1,), (1,)), ((), ())),
            preferred_element_type=jnp.float32,
        )
        + jax.lax.dot_general(
            Tx1, w1_ref[...], (((1,), (1,)), ((), ())),
            preferred_element_type=jnp.float32,
        )
        + b_ref[...]
    )
    out_ref[...] = out


def kernel(x, W0, W1, b):
    Bn, Cn, Hn, Wn = x.shape
    n = Bn * Hn * Wn
    x_f = jnp.transpose(x, (0, 2, 3, 1)).reshape(n, Cn)
    out = pl.pallas_call(
        _body,
        grid=(Bn,),
        in_specs=[
            pl.BlockSpec((_BLK, Cn), lambda i: (i, 0)),
            pl.BlockSpec((Cn, Cn), lambda i: (0, 0)),
            pl.BlockSpec((Cn, Cn), lambda i: (0, 0)),
            pl.BlockSpec((1, Cn), lambda i: (0, 0)),
        ],
        out_specs=pl.BlockSpec((_BLK, Cn), lambda i: (i, 0)),
        out_shape=jax.ShapeDtypeStruct((n, Cn), jnp.float32),
    )(x_f, W0, W1, b.reshape(1, Cn))
    return out


# speedup vs baseline: 1.4476x; 1.0561x over previous
import jax
import jax.numpy as jnp
from jax.experimental import pallas as pl

_BLK = 1024
_K = 9


def _body(x_ref, w0_ref, w1_ref, b_ref, out_ref):
    s = pl.program_id(0)
    X2 = x_ref[...]
    XA = X2[:_BLK]
    XB = X2[_BLK:]

    GA = jax.lax.dot_general(
        XA, XA, (((1,), (1,)), ((), ())), preferred_element_type=jnp.float32
    )
    GB = jax.lax.dot_general(
        XB, XB, (((1,), (1,)), ((), ())), preferred_element_type=jnp.float32
    )

    ii = jax.lax.broadcasted_iota(jnp.int32, (_BLK, _BLK), 0)
    jj = jax.lax.broadcasted_iota(jnp.int32, (_BLK, _BLK), 1)

    def process(X, G, is_last):
        sq = jnp.sum(X * X, axis=1)
        d2 = sq[:, None] + sq[None, :] - 2.0 * G

        cross = jnp.logical_and(
            is_last, jnp.logical_xor(ii == _BLK - 1, jj == _BLK - 1)
        )
        d2 = jnp.where(cross, jnp.inf, d2)

        m = jnp.full((_BLK, 1), -jnp.inf, dtype=jnp.float32)
        for _ in range(_K):
            m = jnp.min(jnp.where(d2 > m, d2, jnp.inf), axis=1, keepdims=True)

        dead = jnp.logical_or(cross, ii == jj)
        Af = jnp.where(
            jnp.logical_and(d2 <= m, jnp.logical_not(dead)), 1.0, 0.0
        )

        deg = jnp.sum(Af, axis=0)
        dinv = jnp.where(deg > 0, jax.lax.rsqrt(jnp.maximum(deg, 1e-12)), 0.0)
        An = (-dinv[:, None] * Af) * dinv[None, :]

        Tx1 = jax.lax.dot_general(
            An, X, (((1,), (0,)), ((), ())), preferred_element_type=jnp.float32
        )
        return (
            jax.lax.dot_general(
                X, w0_ref[...], (((1,), (1,)), ((), ())),
                preferred_element_type=jnp.float32,
            )
            + jax.lax.dot_general(
                Tx1, w1_ref[...], (((1,), (1,)), ((), ())),
                preferred_element_type=jnp.float32,
            )
            + b_ref[...]
        )

    out_ref[:_BLK, :] = process(XA, GA, jnp.bool_(False))
    out_ref[_BLK:, :] = process(XB, GB, s == 1)


def kernel(x, W0, W1, b):
    Bn, Cn, Hn, Wn = x.shape
    n = Bn * Hn * Wn
    x_f = jnp.transpose(x, (0, 2, 3, 1)).reshape(n, Cn)
    out = pl.pallas_call(
        _body,
        grid=(Bn // 2,),
        in_specs=[
            pl.BlockSpec((2 * _BLK, Cn), lambda i: (i, 0)),
            pl.BlockSpec((Cn, Cn), lambda i: (0, 0)),
            pl.BlockSpec((Cn, Cn), lambda i: (0, 0)),
            pl.BlockSpec((1, Cn), lambda i: (0, 0)),
        ],
        out_specs=pl.BlockSpec((2 * _BLK, Cn), lambda i: (i, 0)),
        out_shape=jax.ShapeDtypeStruct((n, Cn), jnp.float32),
    )(x_f, W0, W1, b.reshape(1, Cn))
    return out


# speedup vs baseline: 1.4637x; 1.0111x over previous
import jax
import jax.numpy as jnp
from jax.experimental import pallas as pl

_BLK = 1024
_K = 9


def _body(x_ref, w0_ref, w1_ref, b_ref, out_ref):
    s = pl.program_id(0)
    X2 = x_ref[...]
    XA = X2[:_BLK]
    XB = X2[_BLK:]

    GA = jax.lax.dot_general(
        XA, XA, (((1,), (1,)), ((), ())), preferred_element_type=jnp.float32
    )
    GB = jax.lax.dot_general(
        XB, XB, (((1,), (1,)), ((), ())), preferred_element_type=jnp.float32
    )

    ii = jax.lax.broadcasted_iota(jnp.int32, (_BLK, _BLK), 0)
    jj = jax.lax.broadcasted_iota(jnp.int32, (_BLK, _BLK), 1)

    def process(X, G, is_last):
        sq = jnp.sum(X * X, axis=1)
        d2 = sq[:, None] + sq[None, :] - 2.0 * G

        cross = jnp.logical_and(
            is_last, jnp.logical_xor(ii == _BLK - 1, jj == _BLK - 1)
        )
        d2 = jnp.where(cross, jnp.inf, d2)

        m = jnp.full((_BLK, 1), -jnp.inf, dtype=jnp.float32)
        for _ in range(_K):
            m = jnp.min(jnp.where(d2 > m, d2, jnp.inf), axis=1, keepdims=True)

        dead = jnp.logical_or(cross, ii == jj)
        Af = jnp.where(
            jnp.logical_and(d2 <= m, jnp.logical_not(dead)), 1.0, 0.0
        )

        deg = jnp.sum(Af, axis=0)
        dinv = jnp.where(deg > 0, jax.lax.rsqrt(jnp.maximum(deg, 1e-12)), 0.0)
        An = Af * dinv[None, :]

        Tx1 = -dinv[:, None] * jax.lax.dot_general(
            An, X, (((1,), (0,)), ((), ())), preferred_element_type=jnp.float32
        )
        return (
            jax.lax.dot_general(
                X, w0_ref[...], (((1,), (1,)), ((), ())),
                preferred_element_type=jnp.float32,
            )
            + jax.lax.dot_general(
                Tx1, w1_ref[...], (((1,), (1,)), ((), ())),
                preferred_element_type=jnp.float32,
            )
            + b_ref[...]
        )

    out_ref[:_BLK, :] = process(XA, GA, jnp.bool_(False))
    out_ref[_BLK:, :] = process(XB, GB, s == 1)


def kernel(x, W0, W1, b):
    Bn, Cn, Hn, Wn = x.shape
    n = Bn * Hn * Wn
    x_f = jnp.transpose(x, (0, 2, 3, 1)).reshape(n, Cn)
    out = pl.pallas_call(
        _body,
        grid=(Bn // 2,),
        in_specs=[
            pl.BlockSpec((2 * _BLK, Cn), lambda i: (i, 0)),
            pl.BlockSpec((Cn, Cn), lambda i: (0, 0)),
            pl.BlockSpec((Cn, Cn), lambda i: (0, 0)),
            pl.BlockSpec((1, Cn), lambda i: (0, 0)),
        ],
        out_specs=pl.BlockSpec((2 * _BLK, Cn), lambda i: (i, 0)),
        out_shape=jax.ShapeDtypeStruct((n, Cn), jnp.float32),
    )(x_f, W0, W1, b.reshape(1, Cn))
    return out
